# baseline (speedup 1.0000x reference)
import functools

import jax
import jax.numpy as jnp
from jax import lax
from jax.experimental import pallas as pl
from jax.experimental.pallas import tpu as pltpu
from jax.experimental.pallas import tpu_sc as plsc

N = 10000
E = 320000
D_IN = 128
D_HID = 64

NC = 2
NS = 16
NW = NC * NS
BB = 128
NB = 79
EPT = NB * BB
E_PAD = EPT * NW
N_ACC = 10240
ZSL = N_ACC // BB // NS
R_DUMP = N_ACC // NS
ROW_BLK = 2000


def _mesh():
    return plsc.VectorSubcoreMesh(core_axis_name="c", subcore_axis_name="s")


@functools.partial(
    pl.kernel,
    out_type=jax.ShapeDtypeStruct((NC, N_ACC), jnp.float32),
    mesh=_mesh(),
    scratch_types=[
        pltpu.VMEM((NB, BB), jnp.int32),
        pltpu.VMEM((BB,), jnp.float32),
        pltpu.VMEM((BB,), jnp.float32),
        pltpu.VMEM_SHARED((N_ACC,), jnp.float32),
    ],
)
def _sc_degree(dst_hbm, ones_hbm, zeros_hbm, deg_hbm, dst_v, ones_v, zeros_v, acc):
    cid = lax.axis_index("c")
    sid = lax.axis_index("s")
    wid = sid * NC + cid
    pltpu.sync_copy(dst_hbm.at[wid], dst_v)
    pltpu.sync_copy(ones_hbm, ones_v)
    pltpu.sync_copy(zeros_hbm, zeros_v)
    for k in range(ZSL):
        pltpu.sync_copy(zeros_v, acc.at[pl.ds((sid * ZSL + k) * BB, BB)])
    plsc.subcore_barrier()

    def body(j, carry):
        pltpu.sync_copy(ones_v, acc.at[dst_v.at[j]], add=True)
        return carry

    lax.fori_loop(0, NB, body, 0)
    plsc.subcore_barrier()
    pltpu.sync_copy(acc.at[pl.ds(sid * R_DUMP, R_DUMP)],
                    deg_hbm.at[cid, pl.ds(sid * R_DUMP, R_DUMP)])


def _make_scatter(D, nb0, nb1):
    @functools.partial(
        pl.kernel,
        out_type=jax.ShapeDtypeStruct((NC, N_ACC, D), jnp.float32),
        mesh=_mesh(),
        compiler_params=pltpu.CompilerParams(use_tc_tiling_on_sc=False),
        scratch_types=[
            pltpu.VMEM((2, BB), jnp.int32),
            pltpu.VMEM((2, BB), jnp.int32),
            pltpu.VMEM((BB, D), jnp.float32),
            pltpu.VMEM((BB, D), jnp.float32),
            pltpu.VMEM_SHARED((N_ACC, D), jnp.float32),
            pltpu.SemaphoreType.DMA,
            pltpu.SemaphoreType.DMA,
            pltpu.SemaphoreType.DMA,
            pltpu.SemaphoreType.DMA,
        ],
    )
    def _scatter(g_hbm, idx_hbm, zrows_hbm, part_hbm,
                 ib0, ib1, buf0, buf1, acc, semg0, semg1, semi0, semi1):
        cid = lax.axis_index("c")
        sid = lax.axis_index("s")
        nb = jnp.where(cid == 0, nb0, nb1)
        base = jnp.where(cid == 0, sid * nb0, NS * nb0 + sid * nb1)
        pltpu.sync_copy(zrows_hbm, buf0)
        for k in range(ZSL):
            pltpu.sync_copy(buf0, acc.at[pl.ds((sid * ZSL + k) * BB, BB)])
        plsc.subcore_barrier()

        pltpu.sync_copy(idx_hbm.at[base], ib0)
        pltpu.async_copy(g_hbm.at[ib0.at[0]], buf0, semg0)
        pltpu.async_copy(idx_hbm.at[base + 1], ib1, semi1)

        def step(j, ib_cur, ib_nxt, buf_cur, buf_nxt,
                 semg_cur, semg_nxt, semi_cur, semi_nxt):
            nxt = j + 1

            @pl.when(nxt < nb)
            def _():
                pltpu.make_async_copy(idx_hbm.at[base + nxt], ib_nxt, semi_nxt).wait()
                pltpu.async_copy(g_hbm.at[ib_nxt.at[0]], buf_nxt, semg_nxt)

            pltpu.make_async_copy(g_hbm.at[ib_cur.at[0]], buf_cur, semg_cur).wait()
            pltpu.sync_copy(buf_cur, acc.at[ib_cur.at[1]], add=True)

            @pl.when(j + 2 < nb)
            def _():
                pltpu.async_copy(idx_hbm.at[base + j + 2], ib_cur, semi_cur)

        def body(j, carry):
            @pl.when(j % 2 == 0)
            def _():
                step(j, ib0, ib1, buf0, buf1, semg0, semg1, semi0, semi1)

            @pl.when(j % 2 == 1)
            def _():
                step(j, ib1, ib0, buf1, buf0, semg1, semg0, semi1, semi0)

            return carry

        lax.fori_loop(0, nb, body, 0)
        plsc.subcore_barrier()
        pltpu.sync_copy(acc.at[pl.ds(sid * R_DUMP, R_DUMP)],
                        part_hbm.at[cid, pl.ds(sid * R_DUMP, R_DUMP)])

    return _scatter


TB = NB * NW
_sc_scatter64 = _make_scatter(D_HID, 63, 95)
_sc_scatter128 = _make_scatter(D_IN, 54, 104)


def _dinv_from(degT_ref):
    deg = degT_ref[...].sum(axis=1, keepdims=True) + 1.0
    return lax.rsqrt(jnp.maximum(deg, 1.0))


def _tc1_body(degT_ref, x_ref, W1_ref, g1_ref):
    dinv = _dinv_from(degT_ref)
    g1_ref[...] = jnp.dot(x_ref[...], W1_ref[...],
                          preferred_element_type=jnp.float32) * dinv


def _tc2_body(degT_ref, p_ref, g1_ref, b1_ref, W2_ref, g2_ref):
    dinv = _dinv_from(degT_ref)
    s = p_ref[0] + p_ref[1] + g1_ref[...]
    h = jnp.maximum(dinv * s + b1_ref[...], 0.0)
    g2_ref[...] = jnp.dot(h, W2_ref[...],
                          preferred_element_type=jnp.float32) * dinv


def _tc3_body(degT_ref, q_ref, g2_ref, b2_ref, out_ref):
    dinv = _dinv_from(degT_ref)
    out_ref[...] = dinv * (q_ref[0] + q_ref[1] + g2_ref[...]) + b2_ref[...]


_GRID = (N // ROW_BLK,)

_tc1 = pl.pallas_call(
    _tc1_body,
    grid=_GRID,
    in_specs=[
        pl.BlockSpec((ROW_BLK, 2), lambda i: (i, 0)),
        pl.BlockSpec((ROW_BLK, D_IN), lambda i: (i, 0)),
        pl.BlockSpec((D_IN, D_HID), lambda i: (0, 0)),
    ],
    out_specs=pl.BlockSpec((ROW_BLK, D_HID), lambda i: (i, 0)),
    out_shape=jax.ShapeDtypeStruct((N, D_HID), jnp.float32),
)

_tc2 = pl.pallas_call(
    _tc2_body,
    grid=_GRID,
    in_specs=[
        pl.BlockSpec((ROW_BLK, 2), lambda i: (i, 0)),
        pl.BlockSpec((NC, ROW_BLK, D_HID), lambda i: (0, i, 0)),
        pl.BlockSpec((ROW_BLK, D_HID), lambda i: (i, 0)),
        pl.BlockSpec((1, D_HID), lambda i: (0, 0)),
        pl.BlockSpec((D_HID, D_IN), lambda i: (0, 0)),
    ],
    out_specs=pl.BlockSpec((ROW_BLK, D_IN), lambda i: (i, 0)),
    out_shape=jax.ShapeDtypeStruct((N, D_IN), jnp.float32),
)

_tc3 = pl.pallas_call(
    _tc3_body,
    grid=_GRID,
    in_specs=[
        pl.BlockSpec((ROW_BLK, 2), lambda i: (i, 0)),
        pl.BlockSpec((NC, ROW_BLK, D_IN), lambda i: (0, i, 0)),
        pl.BlockSpec((ROW_BLK, D_IN), lambda i: (i, 0)),
        pl.BlockSpec((1, D_IN), lambda i: (0, 0)),
    ],
    out_specs=pl.BlockSpec((ROW_BLK, D_IN), lambda i: (i, 0)),
    out_shape=jax.ShapeDtypeStruct((N, D_IN), jnp.float32),
)


def kernel(x, edge_index, W1, b1, W2, b2):
    src = edge_index[0].astype(jnp.int32)
    dst = edge_index[1].astype(jnp.int32)
    pad = E_PAD - E
    srcp = jnp.concatenate([src, jnp.zeros((pad,), jnp.int32)]).reshape(NW, NB, BB)
    dstp = jnp.concatenate([dst, jnp.full((pad,), N, jnp.int32)]).reshape(NW, NB, BB)
    ones = jnp.ones((BB,), jnp.float32)
    zeros = jnp.zeros((BB,), jnp.float32)
    z64 = jnp.zeros((BB, D_HID), jnp.float32)
    z128 = jnp.zeros((BB, D_IN), jnp.float32)

    idx = jnp.stack([srcp.reshape(TB, BB), dstp.reshape(TB, BB)], axis=1)

    deg2 = _sc_degree(dstp, ones, zeros)
    degT = deg2.T
    g1 = _tc1(degT, x, W1)
    p = _sc_scatter64(g1, idx, z64)
    g2 = _tc2(degT, p, g1, b1.reshape(1, -1), W2)
    q = _sc_scatter128(g2, idx, z128)
    out = _tc3(degT, q, g2, b2.reshape(1, -1))
    return out


# speedup vs baseline: 1.1063x; 1.1063x over previous
import functools

import jax
import jax.numpy as jnp
from jax import lax
from jax.experimental import pallas as pl
from jax.experimental.pallas import tpu as pltpu
from jax.experimental.pallas import tpu_sc as plsc

N = 10000
E = 320000
D_IN = 128
D_HID = 64

NC = 2
NS = 16
NW = NC * NS
BB = 128
NB = 79
EPT = NB * BB
E_PAD = EPT * NW
N_ACC = 10240
ZSL = N_ACC // BB // NS
R_DUMP = N_ACC // NS
ROW_BLK = 2000


def _mesh():
    return plsc.VectorSubcoreMesh(core_axis_name="c", subcore_axis_name="s")


@functools.partial(
    pl.kernel,
    out_type=jax.ShapeDtypeStruct((NC, N_ACC), jnp.float32),
    mesh=_mesh(),
    scratch_types=[
        pltpu.VMEM((NB, BB), jnp.int32),
        pltpu.VMEM((BB,), jnp.float32),
        pltpu.VMEM((BB,), jnp.float32),
        pltpu.VMEM_SHARED((N_ACC,), jnp.float32),
    ],
)
def _sc_degree(dst_hbm, ones_hbm, zeros_hbm, deg_hbm, dst_v, ones_v, zeros_v, acc):
    cid = lax.axis_index("c")
    sid = lax.axis_index("s")
    wid = sid * NC + cid
    pltpu.sync_copy(dst_hbm.at[wid], dst_v)
    pltpu.sync_copy(ones_hbm, ones_v)
    pltpu.sync_copy(zeros_hbm, zeros_v)
    for k in range(ZSL):
        pltpu.sync_copy(zeros_v, acc.at[pl.ds((sid * ZSL + k) * BB, BB)])
    plsc.subcore_barrier()

    def body(j, carry):
        pltpu.sync_copy(ones_v, acc.at[dst_v.at[j]], add=True)
        return carry

    lax.fori_loop(0, NB, body, 0)
    plsc.subcore_barrier()
    pltpu.sync_copy(acc.at[pl.ds(sid * R_DUMP, R_DUMP)],
                    deg_hbm.at[cid, pl.ds(sid * R_DUMP, R_DUMP)])


def _make_scatter(D, nb0, nb1):
    @functools.partial(
        pl.kernel,
        out_type=jax.ShapeDtypeStruct((NC, N_ACC, D), jnp.float32),
        mesh=_mesh(),
        compiler_params=pltpu.CompilerParams(use_tc_tiling_on_sc=False),
        scratch_types=[
            pltpu.VMEM((2, BB), jnp.int32),
            pltpu.VMEM((2, BB), jnp.int32),
            pltpu.VMEM((BB, D), jnp.float32),
            pltpu.VMEM((BB, D), jnp.float32),
            pltpu.VMEM_SHARED((N_ACC, D), jnp.float32),
            pltpu.SemaphoreType.DMA,
            pltpu.SemaphoreType.DMA,
            pltpu.SemaphoreType.DMA,
            pltpu.SemaphoreType.DMA,
        ],
    )
    def _scatter(g_hbm, idx_hbm, zrows_hbm, part_hbm,
                 ib0, ib1, buf0, buf1, acc, semg0, semg1, semi0, semi1):
        cid = lax.axis_index("c")
        sid = lax.axis_index("s")
        nb = jnp.where(cid == 0, nb0, nb1)
        base = jnp.where(cid == 0, sid * nb0, NS * nb0 + sid * nb1)
        pltpu.sync_copy(zrows_hbm, buf0)
        for k in range(ZSL):
            pltpu.sync_copy(buf0, acc.at[pl.ds((sid * ZSL + k) * BB, BB)])
        plsc.subcore_barrier()

        pltpu.sync_copy(idx_hbm.at[base], ib0)
        pltpu.async_copy(g_hbm.at[ib0.at[0]], buf0, semg0)
        pltpu.async_copy(idx_hbm.at[base + 1], ib1, semi1)

        def step(j, ib_cur, ib_nxt, buf_cur, buf_nxt,
                 semg_cur, semg_nxt, semi_cur, semi_nxt):
            nxt = j + 1

            @pl.when(nxt < nb)
            def _():
                pltpu.make_async_copy(idx_hbm.at[base + nxt], ib_nxt, semi_nxt).wait()
                pltpu.async_copy(g_hbm.at[ib_nxt.at[0]], buf_nxt, semg_nxt)

            pltpu.make_async_copy(g_hbm.at[ib_cur.at[0]], buf_cur, semg_cur).wait()
            pltpu.sync_copy(buf_cur, acc.at[ib_cur.at[1]], add=True)

            @pl.when(j + 2 < nb)
            def _():
                pltpu.async_copy(idx_hbm.at[base + j + 2], ib_cur, semi_cur)

        def body(j, carry):
            @pl.when(j % 2 == 0)
            def _():
                step(j, ib0, ib1, buf0, buf1, semg0, semg1, semi0, semi1)

            @pl.when(j % 2 == 1)
            def _():
                step(j, ib1, ib0, buf1, buf0, semg1, semg0, semi1, semi0)

            return carry

        lax.fori_loop(0, nb, body, 0)
        plsc.subcore_barrier()
        pltpu.sync_copy(acc.at[pl.ds(sid * R_DUMP, R_DUMP)],
                        part_hbm.at[cid, pl.ds(sid * R_DUMP, R_DUMP)])

    return _scatter


TB = NB * NW
_sc_scatter64 = _make_scatter(D_HID, 95, 63)
_sc_scatter128 = _make_scatter(D_IN, 104, 54)


def _dinv_from(degT_ref):
    deg = degT_ref[...].sum(axis=1, keepdims=True) + 1.0
    return lax.rsqrt(jnp.maximum(deg, 1.0))


def _tc1_body(degT_ref, x_ref, W1_ref, g1_ref):
    dinv = _dinv_from(degT_ref)
    g1_ref[...] = jnp.dot(x_ref[...], W1_ref[...],
                          preferred_element_type=jnp.float32) * dinv


def _tc2_body(degT_ref, p_ref, g1_ref, b1_ref, W2_ref, g2_ref):
    dinv = _dinv_from(degT_ref)
    s = p_ref[0] + p_ref[1] + g1_ref[...]
    h = jnp.maximum(dinv * s + b1_ref[...], 0.0)
    g2_ref[...] = jnp.dot(h, W2_ref[...],
                          preferred_element_type=jnp.float32) * dinv


def _tc3_body(degT_ref, q_ref, g2_ref, b2_ref, out_ref):
    dinv = _dinv_from(degT_ref)
    out_ref[...] = dinv * (q_ref[0] + q_ref[1] + g2_ref[...]) + b2_ref[...]


_GRID = (N // ROW_BLK,)

_tc1 = pl.pallas_call(
    _tc1_body,
    grid=_GRID,
    in_specs=[
        pl.BlockSpec((ROW_BLK, 2), lambda i: (i, 0)),
        pl.BlockSpec((ROW_BLK, D_IN), lambda i: (i, 0)),
        pl.BlockSpec((D_IN, D_HID), lambda i: (0, 0)),
    ],
    out_specs=pl.BlockSpec((ROW_BLK, D_HID), lambda i: (i, 0)),
    out_shape=jax.ShapeDtypeStruct((N, D_HID), jnp.float32),
)

_tc2 = pl.pallas_call(
    _tc2_body,
    grid=_GRID,
    in_specs=[
        pl.BlockSpec((ROW_BLK, 2), lambda i: (i, 0)),
        pl.BlockSpec((NC, ROW_BLK, D_HID), lambda i: (0, i, 0)),
        pl.BlockSpec((ROW_BLK, D_HID), lambda i: (i, 0)),
        pl.BlockSpec((1, D_HID), lambda i: (0, 0)),
        pl.BlockSpec((D_HID, D_IN), lambda i: (0, 0)),
    ],
    out_specs=pl.BlockSpec((ROW_BLK, D_IN), lambda i: (i, 0)),
    out_shape=jax.ShapeDtypeStruct((N, D_IN), jnp.float32),
)

_tc3 = pl.pallas_call(
    _tc3_body,
    grid=_GRID,
    in_specs=[
        pl.BlockSpec((ROW_BLK, 2), lambda i: (i, 0)),
        pl.BlockSpec((NC, ROW_BLK, D_IN), lambda i: (0, i, 0)),
        pl.BlockSpec((ROW_BLK, D_IN), lambda i: (i, 0)),
        pl.BlockSpec((1, D_IN), lambda i: (0, 0)),
    ],
    out_specs=pl.BlockSpec((ROW_BLK, D_IN), lambda i: (i, 0)),
    out_shape=jax.ShapeDtypeStruct((N, D_IN), jnp.float32),
)


def kernel(x, edge_index, W1, b1, W2, b2):
    src = edge_index[0].astype(jnp.int32)
    dst = edge_index[1].astype(jnp.int32)
    pad = E_PAD - E
    srcp = jnp.concatenate([src, jnp.zeros((pad,), jnp.int32)]).reshape(NW, NB, BB)
    dstp = jnp.concatenate([dst, jnp.full((pad,), N, jnp.int32)]).reshape(NW, NB, BB)
    ones = jnp.ones((BB,), jnp.float32)
    zeros = jnp.zeros((BB,), jnp.float32)
    z64 = jnp.zeros((BB, D_HID), jnp.float32)
    z128 = jnp.zeros((BB, D_IN), jnp.float32)

    idx = jnp.stack([srcp.reshape(TB, BB), dstp.reshape(TB, BB)], axis=1)

    deg2 = _sc_degree(dstp, ones, zeros)
    degT = deg2.T
    g1 = _tc1(degT, x, W1)
    p = _sc_scatter64(g1, idx, z64)
    g2 = _tc2(degT, p, g1, b1.reshape(1, -1), W2)
    q = _sc_scatter128(g2, idx, z128)
    out = _tc3(degT, q, g2, b2.reshape(1, -1))
    return out


# speedup vs baseline: 1.1637x; 1.0519x over previous
import functools

import jax
import jax.numpy as jnp
from jax import lax
from jax.experimental import pallas as pl
from jax.experimental.pallas import tpu as pltpu
from jax.experimental.pallas import tpu_sc as plsc

N = 10000
E = 320000
D_IN = 128
D_HID = 64

NC = 2
NS = 16
NW = NC * NS
BB = 128
NB = 79
EPT = NB * BB
E_PAD = EPT * NW
N_ACC = 10240
ZSL = N_ACC // BB // NS
R_DUMP = N_ACC // NS
ROW_BLK = 2000


def _mesh():
    return plsc.VectorSubcoreMesh(core_axis_name="c", subcore_axis_name="s")


@functools.partial(
    pl.kernel,
    out_type=jax.ShapeDtypeStruct((NC, N_ACC), jnp.float32),
    mesh=_mesh(),
    scratch_types=[
        pltpu.VMEM((NB, BB), jnp.int32),
        pltpu.VMEM((BB,), jnp.float32),
        pltpu.VMEM((BB,), jnp.float32),
        pltpu.VMEM_SHARED((N_ACC,), jnp.float32),
    ],
)
def _sc_degree(dst_hbm, ones_hbm, zeros_hbm, deg_hbm, dst_v, ones_v, zeros_v, acc):
    cid = lax.axis_index("c")
    sid = lax.axis_index("s")
    wid = sid * NC + cid
    pltpu.sync_copy(dst_hbm.at[wid], dst_v)
    pltpu.sync_copy(ones_hbm, ones_v)
    pltpu.sync_copy(zeros_hbm, zeros_v)
    for k in range(ZSL):
        pltpu.sync_copy(zeros_v, acc.at[pl.ds((sid * ZSL + k) * BB, BB)])
    plsc.subcore_barrier()

    def body(j, carry):
        pltpu.sync_copy(ones_v, acc.at[dst_v.at[j]], add=True)
        return carry

    lax.fori_loop(0, NB, body, 0)
    plsc.subcore_barrier()
    pltpu.sync_copy(acc.at[pl.ds(sid * R_DUMP, R_DUMP)],
                    deg_hbm.at[cid, pl.ds(sid * R_DUMP, R_DUMP)])


def _make_scatter_spmem(P, nb0, nb1):
    DW = D_HID
    NST = N // NS
    out_types = [jax.ShapeDtypeStruct((NC, N_ACC, DW), jnp.float32)
                 for _ in range(P)]

    @functools.partial(
        pl.kernel,
        out_type=tuple(out_types),
        mesh=_mesh(),
        compiler_params=pltpu.CompilerParams(use_tc_tiling_on_sc=False),
        scratch_types=[
            pltpu.VMEM((2, BB), jnp.int32),
            pltpu.VMEM((2, BB), jnp.int32),
            pltpu.VMEM((BB, DW), jnp.float32),
            pltpu.VMEM((BB, DW), jnp.float32),
            pltpu.VMEM_SHARED((N, DW), jnp.float32),
            pltpu.VMEM_SHARED((N_ACC, DW), jnp.float32),
            pltpu.SemaphoreType.DMA,
            pltpu.SemaphoreType.DMA,
            pltpu.SemaphoreType.DMA,
            pltpu.SemaphoreType.DMA,
        ],
    )
    def _scatter(*refs):
        g_hbms = refs[:P]
        idx_hbm, zrows_hbm = refs[P], refs[P + 1]
        part_hbms = refs[P + 2:2 * P + 2]
        (ib0, ib1, buf0, buf1, gsp, acc,
         semg0, semg1, semi0, semi1) = refs[2 * P + 2:]
        cid = lax.axis_index("c")
        sid = lax.axis_index("s")
        nb = jnp.where(cid == 0, nb0, nb1)
        base = jnp.where(cid == 0, sid * nb0, NS * nb0 + sid * nb1)

        for p in range(P):
            pltpu.sync_copy(g_hbms[p].at[pl.ds(sid * NST, NST)],
                            gsp.at[pl.ds(sid * NST, NST)])
            pltpu.sync_copy(zrows_hbm, buf0)
            for k in range(ZSL):
                pltpu.sync_copy(buf0, acc.at[pl.ds((sid * ZSL + k) * BB, BB)])
            plsc.subcore_barrier()

            pltpu.sync_copy(idx_hbm.at[base], ib0)
            pltpu.async_copy(gsp.at[ib0.at[0]], buf0, semg0)
            pltpu.async_copy(idx_hbm.at[base + 1], ib1, semi1)

            def step(j, ib_cur, ib_nxt, buf_cur, buf_nxt,
                     semg_cur, semg_nxt, semi_cur, semi_nxt):
                nxt = j + 1

                @pl.when(nxt < nb)
                def _():
                    pltpu.make_async_copy(idx_hbm.at[base + nxt], ib_nxt,
                                          semi_nxt).wait()
                    pltpu.async_copy(gsp.at[ib_nxt.at[0]], buf_nxt, semg_nxt)

                pltpu.make_async_copy(gsp.at[ib_cur.at[0]], buf_cur,
                                      semg_cur).wait()
                pltpu.sync_copy(buf_cur, acc.at[ib_cur.at[1]], add=True)

                @pl.when(j + 2 < nb)
                def _():
                    pltpu.async_copy(idx_hbm.at[base + j + 2], ib_cur, semi_cur)

            def body(j, carry):
                @pl.when(j % 2 == 0)
                def _():
                    step(j, ib0, ib1, buf0, buf1, semg0, semg1, semi0, semi1)

                @pl.when(j % 2 == 1)
                def _():
                    step(j, ib1, ib0, buf1, buf0, semg1, semg0, semi1, semi0)

                return carry

            lax.fori_loop(0, nb, body, 0)
            plsc.subcore_barrier()
            pltpu.sync_copy(acc.at[pl.ds(sid * R_DUMP, R_DUMP)],
                            part_hbms[p].at[cid, pl.ds(sid * R_DUMP, R_DUMP)])
            if p + 1 < P:
                plsc.subcore_barrier()

    return _scatter


def _make_scatter(D, nb0, nb1):
    @functools.partial(
        pl.kernel,
        out_type=jax.ShapeDtypeStruct((NC, N_ACC, D), jnp.float32),
        mesh=_mesh(),
        compiler_params=pltpu.CompilerParams(use_tc_tiling_on_sc=False),
        scratch_types=[
            pltpu.VMEM((2, BB), jnp.int32),
            pltpu.VMEM((2, BB), jnp.int32),
            pltpu.VMEM((BB, D), jnp.float32),
            pltpu.VMEM((BB, D), jnp.float32),
            pltpu.VMEM_SHARED((N_ACC, D), jnp.float32),
            pltpu.SemaphoreType.DMA,
            pltpu.SemaphoreType.DMA,
            pltpu.SemaphoreType.DMA,
            pltpu.SemaphoreType.DMA,
        ],
    )
    def _scatter(g_hbm, idx_hbm, zrows_hbm, part_hbm,
                 ib0, ib1, buf0, buf1, acc, semg0, semg1, semi0, semi1):
        cid = lax.axis_index("c")
        sid = lax.axis_index("s")
        nb = jnp.where(cid == 0, nb0, nb1)
        base = jnp.where(cid == 0, sid * nb0, NS * nb0 + sid * nb1)
        pltpu.sync_copy(zrows_hbm, buf0)
        for k in range(ZSL):
            pltpu.sync_copy(buf0, acc.at[pl.ds((sid * ZSL + k) * BB, BB)])
        plsc.subcore_barrier()

        pltpu.sync_copy(idx_hbm.at[base], ib0)
        pltpu.async_copy(g_hbm.at[ib0.at[0]], buf0, semg0)
        pltpu.async_copy(idx_hbm.at[base + 1], ib1, semi1)

        def step(j, ib_cur, ib_nxt, buf_cur, buf_nxt,
                 semg_cur, semg_nxt, semi_cur, semi_nxt):
            nxt = j + 1

            @pl.when(nxt < nb)
            def _():
                pltpu.make_async_copy(idx_hbm.at[base + nxt], ib_nxt, semi_nxt).wait()
                pltpu.async_copy(g_hbm.at[ib_nxt.at[0]], buf_nxt, semg_nxt)

            pltpu.make_async_copy(g_hbm.at[ib_cur.at[0]], buf_cur, semg_cur).wait()
            pltpu.sync_copy(buf_cur, acc.at[ib_cur.at[1]], add=True)

            @pl.when(j + 2 < nb)
            def _():
                pltpu.async_copy(idx_hbm.at[base + j + 2], ib_cur, semi_cur)

        def body(j, carry):
            @pl.when(j % 2 == 0)
            def _():
                step(j, ib0, ib1, buf0, buf1, semg0, semg1, semi0, semi1)

            @pl.when(j % 2 == 1)
            def _():
                step(j, ib1, ib0, buf1, buf0, semg1, semg0, semi1, semi0)

            return carry

        lax.fori_loop(0, nb, body, 0)
        plsc.subcore_barrier()
        pltpu.sync_copy(acc.at[pl.ds(sid * R_DUMP, R_DUMP)],
                        part_hbm.at[cid, pl.ds(sid * R_DUMP, R_DUMP)])

    return _scatter


TB = NB * NW
_sc_scatter64 = _make_scatter_spmem(1, NB, NB)
_sc_scatter128 = _make_scatter_spmem(2, NB, NB)


def _dinv_from(degT_ref):
    deg = degT_ref[...].sum(axis=1, keepdims=True) + 1.0
    return lax.rsqrt(jnp.maximum(deg, 1.0))


def _tc1_body(degT_ref, x_ref, W1_ref, g1_ref):
    dinv = _dinv_from(degT_ref)
    g1_ref[...] = jnp.dot(x_ref[...], W1_ref[...],
                          preferred_element_type=jnp.float32) * dinv


def _tc2_body(degT_ref, p_ref, g1_ref, b1_ref, W2_ref, g2a_ref, g2b_ref):
    dinv = _dinv_from(degT_ref)
    s = p_ref[0] + p_ref[1] + g1_ref[...]
    h = jnp.maximum(dinv * s + b1_ref[...], 0.0)
    g2 = jnp.dot(h, W2_ref[...], preferred_element_type=jnp.float32) * dinv
    g2a_ref[...] = g2[:, :D_HID]
    g2b_ref[...] = g2[:, D_HID:]


def _tc3_body(degT_ref, qa_ref, qb_ref, g2a_ref, g2b_ref, b2_ref, out_ref):
    dinv = _dinv_from(degT_ref)
    ya = dinv * (qa_ref[0] + qa_ref[1] + g2a_ref[...])
    yb = dinv * (qb_ref[0] + qb_ref[1] + g2b_ref[...])
    out_ref[...] = jnp.concatenate([ya, yb], axis=1) + b2_ref[...]


_GRID = (N // ROW_BLK,)

_tc1 = pl.pallas_call(
    _tc1_body,
    grid=_GRID,
    in_specs=[
        pl.BlockSpec((ROW_BLK, 2), lambda i: (i, 0)),
        pl.BlockSpec((ROW_BLK, D_IN), lambda i: (i, 0)),
        pl.BlockSpec((D_IN, D_HID), lambda i: (0, 0)),
    ],
    out_specs=pl.BlockSpec((ROW_BLK, D_HID), lambda i: (i, 0)),
    out_shape=jax.ShapeDtypeStruct((N, D_HID), jnp.float32),
)

_tc2 = pl.pallas_call(
    _tc2_body,
    grid=_GRID,
    in_specs=[
        pl.BlockSpec((ROW_BLK, 2), lambda i: (i, 0)),
        pl.BlockSpec((NC, ROW_BLK, D_HID), lambda i: (0, i, 0)),
        pl.BlockSpec((ROW_BLK, D_HID), lambda i: (i, 0)),
        pl.BlockSpec((1, D_HID), lambda i: (0, 0)),
        pl.BlockSpec((D_HID, D_IN), lambda i: (0, 0)),
    ],
    out_specs=[
        pl.BlockSpec((ROW_BLK, D_HID), lambda i: (i, 0)),
        pl.BlockSpec((ROW_BLK, D_HID), lambda i: (i, 0)),
    ],
    out_shape=[
        jax.ShapeDtypeStruct((N, D_HID), jnp.float32),
        jax.ShapeDtypeStruct((N, D_HID), jnp.float32),
    ],
)

_tc3 = pl.pallas_call(
    _tc3_body,
    grid=_GRID,
    in_specs=[
        pl.BlockSpec((ROW_BLK, 2), lambda i: (i, 0)),
        pl.BlockSpec((NC, ROW_BLK, D_HID), lambda i: (0, i, 0)),
        pl.BlockSpec((NC, ROW_BLK, D_HID), lambda i: (0, i, 0)),
        pl.BlockSpec((ROW_BLK, D_HID), lambda i: (i, 0)),
        pl.BlockSpec((ROW_BLK, D_HID), lambda i: (i, 0)),
        pl.BlockSpec((1, D_IN), lambda i: (0, 0)),
    ],
    out_specs=pl.BlockSpec((ROW_BLK, D_IN), lambda i: (i, 0)),
    out_shape=jax.ShapeDtypeStruct((N, D_IN), jnp.float32),
)


def kernel(x, edge_index, W1, b1, W2, b2):
    src = edge_index[0].astype(jnp.int32)
    dst = edge_index[1].astype(jnp.int32)
    pad = E_PAD - E
    srcp = jnp.concatenate([src, jnp.zeros((pad,), jnp.int32)]).reshape(NW, NB, BB)
    dstp = jnp.concatenate([dst, jnp.full((pad,), N, jnp.int32)]).reshape(NW, NB, BB)
    ones = jnp.ones((BB,), jnp.float32)
    zeros = jnp.zeros((BB,), jnp.float32)
    z64 = jnp.zeros((BB, D_HID), jnp.float32)

    idx = jnp.stack([srcp.reshape(TB, BB), dstp.reshape(TB, BB)], axis=1)

    deg2 = _sc_degree(dstp, ones, zeros)
    degT = deg2.T
    g1 = _tc1(degT, x, W1)
    (p,) = _sc_scatter64(g1, idx, z64)
    g2a, g2b = _tc2(degT, p, g1, b1.reshape(1, -1), W2)
    qa, qb = _sc_scatter128(g2a, g2b, idx, z64)
    out = _tc3(degT, qa, qb, g2a, g2b, b2.reshape(1, -1))
    return out


# speedup vs baseline: 1.4141x; 1.2152x over previous
import functools

import jax
import jax.numpy as jnp
from jax import lax
from jax.experimental import pallas as pl
from jax.experimental.pallas import tpu as pltpu
from jax.experimental.pallas import tpu_sc as plsc

N = 10000
E = 320000
D_IN = 128
D_HID = 64

NC = 2
NS = 16
NW = NC * NS
BB = 128
NB = 79
EPT = NB * BB
E_PAD = EPT * NW
N_ACC = 10240
ZSL = N_ACC // BB // NS
R_DUMP = N_ACC // NS
ROW_BLK = 2000


def _mesh():
    return plsc.VectorSubcoreMesh(core_axis_name="c", subcore_axis_name="s")


@functools.partial(
    pl.kernel,
    out_type=jax.ShapeDtypeStruct((NC, N_ACC), jnp.float32),
    mesh=_mesh(),
    compiler_params=pltpu.CompilerParams(use_tc_tiling_on_sc=False),
    scratch_types=[
        pltpu.VMEM((NB, 2, BB), jnp.int32),
        pltpu.VMEM((BB,), jnp.float32),
        pltpu.VMEM((BB,), jnp.float32),
        pltpu.VMEM_SHARED((N_ACC,), jnp.float32),
    ],
)
def _sc_degree(idx_hbm, ones_hbm, zeros_hbm, deg_hbm, idx_v, ones_v, zeros_v, acc):
    cid = lax.axis_index("c")
    sid = lax.axis_index("s")
    wid = sid * NC + cid
    pltpu.sync_copy(idx_hbm.at[pl.ds(wid * NB, NB)], idx_v)
    pltpu.sync_copy(ones_hbm, ones_v)
    pltpu.sync_copy(zeros_hbm, zeros_v)
    for k in range(ZSL):
        pltpu.sync_copy(zeros_v, acc.at[pl.ds((sid * ZSL + k) * BB, BB)])
    plsc.subcore_barrier()

    def body(j, carry):
        pltpu.sync_copy(ones_v, acc.at[idx_v.at[j, 1]], add=True)
        return carry

    lax.fori_loop(0, NB, body, 0)
    plsc.subcore_barrier()
    pltpu.sync_copy(acc.at[pl.ds(sid * R_DUMP, R_DUMP)],
                    deg_hbm.at[cid, pl.ds(sid * R_DUMP, R_DUMP)])


def _make_scatter_spmem(P, nb0, nb1):
    DW = D_HID
    NST = N // NS
    out_types = [jax.ShapeDtypeStruct((NC, N_ACC, DW), jnp.float32)
                 for _ in range(P)]

    @functools.partial(
        pl.kernel,
        out_type=tuple(out_types),
        mesh=_mesh(),
        compiler_params=pltpu.CompilerParams(use_tc_tiling_on_sc=False),
        scratch_types=[
            pltpu.VMEM((2, BB), jnp.int32),
            pltpu.VMEM((2, BB), jnp.int32),
            pltpu.VMEM((2, BB), jnp.int32),
            pltpu.VMEM((2, BB), jnp.int32),
            pltpu.VMEM((BB, DW), jnp.float32),
            pltpu.VMEM((BB, DW), jnp.float32),
            pltpu.VMEM_SHARED((N, DW), jnp.float32),
            pltpu.VMEM_SHARED((N_ACC, DW), jnp.float32),
            pltpu.SemaphoreType.DMA,
            pltpu.SemaphoreType.DMA,
            pltpu.SemaphoreType.DMA,
            pltpu.SemaphoreType.DMA,
            pltpu.SemaphoreType.DMA,
            pltpu.SemaphoreType.DMA,
        ],
    )
    def _scatter(*refs):
        g_hbms = refs[:P]
        idx_hbm, zrows_hbm = refs[P], refs[P + 1]
        part_hbms = refs[P + 2:2 * P + 2]
        (ib0, ib1, ib2, ib3, buf0, buf1, gsp, acc,
         semg0, semg1, semi0, semi1, semi2, semi3) = refs[2 * P + 2:]
        ibs = (ib0, ib1, ib2, ib3)
        bufs = (buf0, buf1)
        semgs = (semg0, semg1)
        semis = (semi0, semi1, semi2, semi3)
        cid = lax.axis_index("c")
        sid = lax.axis_index("s")
        nb = jnp.where(cid == 0, nb0, nb1)
        base = jnp.where(cid == 0, sid * nb0, NS * nb0 + sid * nb1)

        for p in range(P):
            pltpu.sync_copy(g_hbms[p].at[pl.ds(sid * NST, NST)],
                            gsp.at[pl.ds(sid * NST, NST)])
            pltpu.sync_copy(zrows_hbm, buf0)
            for k in range(ZSL):
                pltpu.sync_copy(buf0, acc.at[pl.ds((sid * ZSL + k) * BB, BB)])
            plsc.subcore_barrier()

            pltpu.sync_copy(idx_hbm.at[base], ib0)
            pltpu.async_copy(gsp.at[ib0.at[0]], buf0, semg0)
            pltpu.async_copy(idx_hbm.at[base + 1], ib1, semi1)

            @pl.when(2 < nb)
            def _():
                pltpu.async_copy(idx_hbm.at[base + 2], ib2, semi2)

            def step(j, k):
                nxt = j + 1
                k1, k3 = (k + 1) % 4, (k + 3) % 4

                @pl.when(nxt < nb)
                def _():
                    pltpu.make_async_copy(idx_hbm.at[base + nxt], ibs[k1],
                                          semis[k1]).wait()
                    pltpu.async_copy(gsp.at[ibs[k1].at[0]], bufs[k1 % 2],
                                     semgs[k1 % 2])

                pltpu.make_async_copy(gsp.at[ibs[k].at[0]], bufs[k % 2],
                                      semgs[k % 2]).wait()

                @pl.when(j + 3 < nb)
                def _():
                    pltpu.async_copy(idx_hbm.at[base + j + 3], ibs[k3],
                                     semis[k3])

                pltpu.sync_copy(bufs[k % 2], acc.at[ibs[k].at[1]], add=True)

            def body(j, carry):
                for k in range(4):
                    @pl.when(j % 4 == k)
                    def _(k=k):
                        step(j, k)

                return carry

            lax.fori_loop(0, nb, body, 0)
            plsc.subcore_barrier()
            pltpu.sync_copy(acc.at[pl.ds(sid * R_DUMP, R_DUMP)],
                            part_hbms[p].at[cid, pl.ds(sid * R_DUMP, R_DUMP)])
            if p + 1 < P:
                plsc.subcore_barrier()

    return _scatter


TB = NB * NW
_sc_scatter64 = _make_scatter_spmem(1, NB, NB)
_sc_scatter128 = _make_scatter_spmem(2, NB, NB)


def _dinv_from(degT_ref):
    deg = degT_ref[...].sum(axis=1, keepdims=True) + 1.0
    return lax.rsqrt(jnp.maximum(deg, 1.0))


def _tc1a_body(x_ref, W1_ref, mm_ref):
    mm_ref[...] = jnp.dot(x_ref[...], W1_ref[...],
                          preferred_element_type=jnp.float32)


def _tc1b_body(degT_ref, mm_ref, g1_ref):
    g1_ref[...] = mm_ref[...] * _dinv_from(degT_ref)


def _tc2_body(degT_ref, p_ref, g1_ref, b1_ref, W2_ref, g2a_ref, g2b_ref):
    dinv = _dinv_from(degT_ref)
    s = p_ref[0] + p_ref[1] + g1_ref[...]
    h = jnp.maximum(dinv * s + b1_ref[...], 0.0)
    g2 = jnp.dot(h, W2_ref[...], preferred_element_type=jnp.float32) * dinv
    g2a_ref[...] = g2[:, :D_HID]
    g2b_ref[...] = g2[:, D_HID:]


def _tc3_body(degT_ref, qa_ref, qb_ref, g2a_ref, g2b_ref, b2_ref, out_ref):
    dinv = _dinv_from(degT_ref)
    ya = dinv * (qa_ref[0] + qa_ref[1] + g2a_ref[...])
    yb = dinv * (qb_ref[0] + qb_ref[1] + g2b_ref[...])
    out_ref[...] = jnp.concatenate([ya, yb], axis=1) + b2_ref[...]


_GRID = (N // ROW_BLK,)

_tc1a = pl.pallas_call(
    _tc1a_body,
    grid=_GRID,
    in_specs=[
        pl.BlockSpec((ROW_BLK, D_IN), lambda i: (i, 0)),
        pl.BlockSpec((D_IN, D_HID), lambda i: (0, 0)),
    ],
    out_specs=pl.BlockSpec((ROW_BLK, D_HID), lambda i: (i, 0)),
    out_shape=jax.ShapeDtypeStruct((N, D_HID), jnp.float32),
)

_tc1b = pl.pallas_call(
    _tc1b_body,
    grid=_GRID,
    in_specs=[
        pl.BlockSpec((ROW_BLK, 2), lambda i: (i, 0)),
        pl.BlockSpec((ROW_BLK, D_HID), lambda i: (i, 0)),
    ],
    out_specs=pl.BlockSpec((ROW_BLK, D_HID), lambda i: (i, 0)),
    out_shape=jax.ShapeDtypeStruct((N, D_HID), jnp.float32),
)

_tc2 = pl.pallas_call(
    _tc2_body,
    grid=_GRID,
    in_specs=[
        pl.BlockSpec((ROW_BLK, 2), lambda i: (i, 0)),
        pl.BlockSpec((NC, ROW_BLK, D_HID), lambda i: (0, i, 0)),
        pl.BlockSpec((ROW_BLK, D_HID), lambda i: (i, 0)),
        pl.BlockSpec((1, D_HID), lambda i: (0, 0)),
        pl.BlockSpec((D_HID, D_IN), lambda i: (0, 0)),
    ],
    out_specs=[
        pl.BlockSpec((ROW_BLK, D_HID), lambda i: (i, 0)),
        pl.BlockSpec((ROW_BLK, D_HID), lambda i: (i, 0)),
    ],
    out_shape=[
        jax.ShapeDtypeStruct((N, D_HID), jnp.float32),
        jax.ShapeDtypeStruct((N, D_HID), jnp.float32),
    ],
)

_tc3 = pl.pallas_call(
    _tc3_body,
    grid=_GRID,
    in_specs=[
        pl.BlockSpec((ROW_BLK, 2), lambda i: (i, 0)),
        pl.BlockSpec((NC, ROW_BLK, D_HID), lambda i: (0, i, 0)),
        pl.BlockSpec((NC, ROW_BLK, D_HID), lambda i: (0, i, 0)),
        pl.BlockSpec((ROW_BLK, D_HID), lambda i: (i, 0)),
        pl.BlockSpec((ROW_BLK, D_HID), lambda i: (i, 0)),
        pl.BlockSpec((1, D_IN), lambda i: (0, 0)),
    ],
    out_specs=pl.BlockSpec((ROW_BLK, D_IN), lambda i: (i, 0)),
    out_shape=jax.ShapeDtypeStruct((N, D_IN), jnp.float32),
)


def kernel(x, edge_index, W1, b1, W2, b2):
    src = edge_index[0].astype(jnp.int32)
    dst = edge_index[1].astype(jnp.int32)
    pad = E_PAD - E
    srcp = jnp.concatenate([src, jnp.zeros((pad,), jnp.int32)]).reshape(TB, BB)
    dstp = jnp.concatenate([dst, jnp.full((pad,), N, jnp.int32)]).reshape(TB, BB)
    ones = jnp.ones((BB,), jnp.float32)
    zeros = jnp.zeros((BB,), jnp.float32)
    z64 = jnp.zeros((BB, D_HID), jnp.float32)

    idx = jnp.stack([srcp, dstp], axis=1)

    deg2 = _sc_degree(idx, ones, zeros)
    mm1 = _tc1a(x, W1)
    degT = deg2.T
    g1 = _tc1b(degT, mm1)
    (p,) = _sc_scatter64(g1, idx, z64)
    g2a, g2b = _tc2(degT, p, g1, b1.reshape(1, -1), W2)
    qa, qb = _sc_scatter128(g2a, g2b, idx, z64)
    out = _tc3(degT, qa, qb, g2a, g2b, b2.reshape(1, -1))
    return out


# speedup vs baseline: 1.6134x; 1.1409x over previous
import functools

import jax
import jax.numpy as jnp
from jax import lax
from jax.experimental import pallas as pl
from jax.experimental.pallas import tpu as pltpu
from jax.experimental.pallas import tpu_sc as plsc

N = 10000
E = 320000
D_IN = 128
D_HID = 64

NC = 2
NS = 16
NW = NC * NS
BB = 128
NB = 79
EPT = NB * BB
E_PAD = EPT * NW
N_ACC = 10240
ZSL = N_ACC // BB // NS
R_DUMP = N_ACC // NS
ROW_BLK = 2000


def _mesh():
    return plsc.VectorSubcoreMesh(core_axis_name="c", subcore_axis_name="s")


@functools.partial(
    pl.kernel,
    out_type=jax.ShapeDtypeStruct((NC, N_ACC), jnp.float32),
    mesh=_mesh(),
    compiler_params=pltpu.CompilerParams(use_tc_tiling_on_sc=False),
    scratch_types=[
        pltpu.VMEM((NB, 2, BB), jnp.int32),
        pltpu.VMEM((BB,), jnp.float32),
        pltpu.VMEM((BB,), jnp.float32),
        pltpu.VMEM_SHARED((N_ACC,), jnp.float32),
    ],
)
def _sc_degree(idx_hbm, ones_hbm, zeros_hbm, deg_hbm, idx_v, ones_v, zeros_v, acc):
    cid = lax.axis_index("c")
    sid = lax.axis_index("s")
    wid = sid * NC + cid
    pltpu.sync_copy(idx_hbm.at[pl.ds(wid * NB, NB)], idx_v)
    pltpu.sync_copy(ones_hbm, ones_v)
    pltpu.sync_copy(zeros_hbm, zeros_v)
    for k in range(ZSL):
        pltpu.sync_copy(zeros_v, acc.at[pl.ds((sid * ZSL + k) * BB, BB)])
    plsc.subcore_barrier()

    def body(j, carry):
        pltpu.sync_copy(ones_v, acc.at[idx_v.at[j, 1]], add=True)
        return carry

    lax.fori_loop(0, NB, body, 0)
    plsc.subcore_barrier()
    pltpu.sync_copy(acc.at[pl.ds(sid * R_DUMP, R_DUMP)],
                    deg_hbm.at[cid, pl.ds(sid * R_DUMP, R_DUMP)])


def _make_scatter_spmem(P, nb0, nb1):
    DW = D_HID
    NST = N // NS
    out_types = [jax.ShapeDtypeStruct((NC, N_ACC, DW), jnp.float32)
                 for _ in range(P)]

    @functools.partial(
        pl.kernel,
        out_type=tuple(out_types),
        mesh=_mesh(),
        compiler_params=pltpu.CompilerParams(use_tc_tiling_on_sc=False),
        scratch_types=(
            [pltpu.VMEM((2, BB), jnp.int32)] * 8
            + [pltpu.VMEM((BB, DW), jnp.float32)] * 4
            + [
                pltpu.VMEM_SHARED((N, DW), jnp.float32),
                pltpu.VMEM_SHARED((N_ACC, DW), jnp.float32),
            ]
            + [pltpu.SemaphoreType.DMA] * 16
        ),
    )
    def _scatter(*refs):
        g_hbms = refs[:P]
        idx_hbm, zrows_hbm = refs[P], refs[P + 1]
        part_hbms = refs[P + 2:2 * P + 2]
        scr = refs[2 * P + 2:]
        ibs = scr[0:8]
        bufs = scr[8:12]
        gsp, acc = scr[12], scr[13]
        semis = scr[14:22]
        semgs = scr[22:26]
        semss = scr[26:30]
        cid = lax.axis_index("c")
        sid = lax.axis_index("s")
        nb = NB
        base = (cid * NS + sid) * NB

        def gat(b, k4, k8):
            return pltpu.make_async_copy(gsp.at[ibs[k8].at[0]], bufs[k4],
                                         semgs[k4])

        def sca_start(k4, k8):
            pltpu.async_copy(bufs[k4], acc.at[ibs[k8].at[1]], semss[k4],
                             add=True)

        def sca_wait(k4, k8):
            pltpu.make_async_copy(bufs[k4], acc.at[ibs[k8].at[1]],
                                  semss[k4]).wait()

        for p in range(P):
            pltpu.sync_copy(g_hbms[p].at[pl.ds(sid * NST, NST)],
                            gsp.at[pl.ds(sid * NST, NST)])
            pltpu.sync_copy(zrows_hbm, bufs[0])
            for k in range(ZSL):
                pltpu.sync_copy(bufs[0], acc.at[pl.ds((sid * ZSL + k) * BB, BB)])
            plsc.subcore_barrier()

            pltpu.sync_copy(idx_hbm.at[base], ibs[0])
            pltpu.async_copy(idx_hbm.at[base + 1], ibs[1], semis[1])
            pltpu.async_copy(idx_hbm.at[base + 2], ibs[2], semis[2])
            gat(0, 0, 0).start()

            def step(j, k8):
                k4 = k8 % 4
                nk8, nk4 = (k8 + 1) % 8, (k8 + 1) % 4
                nxt = j + 1

                @pl.when(jnp.logical_and(nxt < nb, j >= 3))
                def _():
                    sca_wait(nk4, (k8 + 5) % 8)

                @pl.when(nxt < nb)
                def _():
                    pltpu.make_async_copy(idx_hbm.at[base + nxt], ibs[nk8],
                                          semis[nk8]).wait()
                    gat(nxt, nk4, nk8).start()

                gat(j, k4, k8).wait()

                @pl.when(j + 3 < nb)
                def _():
                    pltpu.async_copy(idx_hbm.at[base + j + 3],
                                     ibs[(k8 + 3) % 8], semis[(k8 + 3) % 8])

                sca_start(k4, k8)

            def body(j, carry):
                for k in range(8):
                    @pl.when(j % 8 == k)
                    def _(k=k):
                        step(j, k)

                return carry

            lax.fori_loop(0, nb, body, 0)
            for t in range(nb - 4, nb):
                sca_wait(t % 4, t % 8)
            plsc.subcore_barrier()
            pltpu.sync_copy(acc.at[pl.ds(sid * R_DUMP, R_DUMP)],
                            part_hbms[p].at[cid, pl.ds(sid * R_DUMP, R_DUMP)])
            if p + 1 < P:
                plsc.subcore_barrier()

    return _scatter


TB = NB * NW
_sc_scatter64 = _make_scatter_spmem(1, NB, NB)
_sc_scatter128 = _make_scatter_spmem(2, NB, NB)


def _dinv_from(degT_ref):
    deg = degT_ref[...].sum(axis=1, keepdims=True) + 1.0
    return lax.rsqrt(jnp.maximum(deg, 1.0))


def _tc1a_body(x_ref, W1_ref, mm_ref):
    mm_ref[...] = jnp.dot(x_ref[...], W1_ref[...],
                          preferred_element_type=jnp.float32)


def _tc1b_body(degT_ref, mm_ref, g1_ref):
    g1_ref[...] = mm_ref[...] * _dinv_from(degT_ref)


def _tc2_body(degT_ref, p_ref, g1_ref, b1_ref, W2_ref, g2a_ref, g2b_ref):
    dinv = _dinv_from(degT_ref)
    s = p_ref[0] + p_ref[1] + g1_ref[...]
    h = jnp.maximum(dinv * s + b1_ref[...], 0.0)
    g2 = jnp.dot(h, W2_ref[...], preferred_element_type=jnp.float32) * dinv
    g2a_ref[...] = g2[:, :D_HID]
    g2b_ref[...] = g2[:, D_HID:]


def _tc3_body(degT_ref, qa_ref, qb_ref, g2a_ref, g2b_ref, b2_ref, out_ref):
    dinv = _dinv_from(degT_ref)
    ya = dinv * (qa_ref[0] + qa_ref[1] + g2a_ref[...])
    yb = dinv * (qb_ref[0] + qb_ref[1] + g2b_ref[...])
    out_ref[...] = jnp.concatenate([ya, yb], axis=1) + b2_ref[...]


_GRID = (N // ROW_BLK,)

_tc1a = pl.pallas_call(
    _tc1a_body,
    grid=_GRID,
    in_specs=[
        pl.BlockSpec((ROW_BLK, D_IN), lambda i: (i, 0)),
        pl.BlockSpec((D_IN, D_HID), lambda i: (0, 0)),
    ],
    out_specs=pl.BlockSpec((ROW_BLK, D_HID), lambda i: (i, 0)),
    out_shape=jax.ShapeDtypeStruct((N, D_HID), jnp.float32),
)

_tc1b = pl.pallas_call(
    _tc1b_body,
    grid=_GRID,
    in_specs=[
        pl.BlockSpec((ROW_BLK, 2), lambda i: (i, 0)),
        pl.BlockSpec((ROW_BLK, D_HID), lambda i: (i, 0)),
    ],
    out_specs=pl.BlockSpec((ROW_BLK, D_HID), lambda i: (i, 0)),
    out_shape=jax.ShapeDtypeStruct((N, D_HID), jnp.float32),
)

_tc2 = pl.pallas_call(
    _tc2_body,
    grid=_GRID,
    in_specs=[
        pl.BlockSpec((ROW_BLK, 2), lambda i: (i, 0)),
        pl.BlockSpec((NC, ROW_BLK, D_HID), lambda i: (0, i, 0)),
        pl.BlockSpec((ROW_BLK, D_HID), lambda i: (i, 0)),
        pl.BlockSpec((1, D_HID), lambda i: (0, 0)),
        pl.BlockSpec((D_HID, D_IN), lambda i: (0, 0)),
    ],
    out_specs=[
        pl.BlockSpec((ROW_BLK, D_HID), lambda i: (i, 0)),
        pl.BlockSpec((ROW_BLK, D_HID), lambda i: (i, 0)),
    ],
    out_shape=[
        jax.ShapeDtypeStruct((N, D_HID), jnp.float32),
        jax.ShapeDtypeStruct((N, D_HID), jnp.float32),
    ],
)

_tc3 = pl.pallas_call(
    _tc3_body,
    grid=_GRID,
    in_specs=[
        pl.BlockSpec((ROW_BLK, 2), lambda i: (i, 0)),
        pl.BlockSpec((NC, ROW_BLK, D_HID), lambda i: (0, i, 0)),
        pl.BlockSpec((NC, ROW_BLK, D_HID), lambda i: (0, i, 0)),
        pl.BlockSpec((ROW_BLK, D_HID), lambda i: (i, 0)),
        pl.BlockSpec((ROW_BLK, D_HID), lambda i: (i, 0)),
        pl.BlockSpec((1, D_IN), lambda i: (0, 0)),
    ],
    out_specs=pl.BlockSpec((ROW_BLK, D_IN), lambda i: (i, 0)),
    out_shape=jax.ShapeDtypeStruct((N, D_IN), jnp.float32),
)


def kernel(x, edge_index, W1, b1, W2, b2):
    src = edge_index[0].astype(jnp.int32)
    dst = edge_index[1].astype(jnp.int32)
    pad = E_PAD - E
    srcp = jnp.concatenate([src, jnp.zeros((pad,), jnp.int32)]).reshape(TB, BB)
    dstp = jnp.concatenate([dst, jnp.full((pad,), N, jnp.int32)]).reshape(TB, BB)
    ones = jnp.ones((BB,), jnp.float32)
    zeros = jnp.zeros((BB,), jnp.float32)
    z64 = jnp.zeros((BB, D_HID), jnp.float32)

    idx = jnp.stack([srcp, dstp], axis=1)

    deg2 = _sc_degree(idx, ones, zeros)
    mm1 = _tc1a(x, W1)
    degT = deg2.T
    g1 = _tc1b(degT, mm1)
    (p,) = _sc_scatter64(g1, idx, z64)
    g2a, g2b = _tc2(degT, p, g1, b1.reshape(1, -1), W2)
    qa, qb = _sc_scatter128(g2a, g2b, idx, z64)
    out = _tc3(degT, qa, qb, g2a, g2b, b2.reshape(1, -1))
    return out


# speedup vs baseline: 1.6816x; 1.0423x over previous
import functools

import jax
import jax.numpy as jnp
from jax import lax
from jax.experimental import pallas as pl
from jax.experimental.pallas import tpu as pltpu
from jax.experimental.pallas import tpu_sc as plsc

N = 10000
E = 320000
D_IN = 128
D_HID = 64

NC = 2
NS = 16
NW = NC * NS
BB = 128
NB = 79
EPT = NB * BB
E_PAD = EPT * NW
N_ACC = 10240
ZSL = N_ACC // BB // NS
R_DUMP = N_ACC // NS
ROW_BLK = 2000


def _mesh():
    return plsc.VectorSubcoreMesh(core_axis_name="c", subcore_axis_name="s")


@functools.partial(
    pl.kernel,
    out_type=jax.ShapeDtypeStruct((NC, N_ACC), jnp.float32),
    mesh=_mesh(),
    compiler_params=pltpu.CompilerParams(use_tc_tiling_on_sc=False),
    scratch_types=[
        pltpu.VMEM((NB, 2, BB), jnp.int32),
        pltpu.VMEM((BB,), jnp.float32),
        pltpu.VMEM((BB,), jnp.float32),
        pltpu.VMEM_SHARED((N_ACC,), jnp.float32),
    ],
)
def _sc_degree(idx_hbm, ones_hbm, zeros_hbm, deg_hbm, idx_v, ones_v, zeros_v, acc):
    cid = lax.axis_index("c")
    sid = lax.axis_index("s")
    wid = sid * NC + cid
    pltpu.sync_copy(idx_hbm.at[pl.ds(wid * NB, NB)], idx_v)
    pltpu.sync_copy(ones_hbm, ones_v)
    pltpu.sync_copy(zeros_hbm, zeros_v)
    for k in range(ZSL):
        pltpu.sync_copy(zeros_v, acc.at[pl.ds((sid * ZSL + k) * BB, BB)])
    plsc.subcore_barrier()

    def body(j, carry):
        pltpu.sync_copy(ones_v, acc.at[idx_v.at[j, 1]], add=True)
        return carry

    lax.fori_loop(0, NB, body, 0)
    plsc.subcore_barrier()
    pltpu.sync_copy(acc.at[pl.ds(sid * R_DUMP, R_DUMP)],
                    deg_hbm.at[cid, pl.ds(sid * R_DUMP, R_DUMP)])


def _make_scatter_spmem(P, nb0, nb1):
    DW = D_HID
    NST = N // NS
    out_types = [jax.ShapeDtypeStruct((NC, N_ACC, DW), jnp.float32)
                 for _ in range(P)]

    @functools.partial(
        pl.kernel,
        out_type=tuple(out_types),
        mesh=_mesh(),
        compiler_params=pltpu.CompilerParams(use_tc_tiling_on_sc=False),
        scratch_types=(
            [pltpu.VMEM((2, BB), jnp.int32)] * 8
            + [pltpu.VMEM((BB, DW), jnp.float32)] * 4
            + [
                pltpu.VMEM_SHARED((N + 16, DW), jnp.float32),
                pltpu.VMEM_SHARED((N_ACC, DW), jnp.float32),
            ]
            + [pltpu.SemaphoreType.DMA] * 16
        ),
    )
    def _scatter(*refs):
        g_hbms = refs[:P]
        idx_hbm, zrows_hbm = refs[P], refs[P + 1]
        part_hbms = refs[P + 2:2 * P + 2]
        scr = refs[2 * P + 2:]
        ibs = scr[0:8]
        bufs = scr[8:12]
        gsp, acc = scr[12], scr[13]
        semis = scr[14:22]
        semgs = scr[22:26]
        semss = scr[26:30]
        cid = lax.axis_index("c")
        sid = lax.axis_index("s")
        nb = NB
        base = (cid * NS + sid) * NB

        def gat(b, k4, k8):
            return pltpu.make_async_copy(gsp.at[ibs[k8].at[0]], bufs[k4],
                                         semgs[k4])

        def sca_start(k4, k8):
            pltpu.async_copy(bufs[k4], acc.at[ibs[k8].at[1]], semss[k4],
                             add=True)

        def sca_wait(k4, k8):
            pltpu.make_async_copy(bufs[k4], acc.at[ibs[k8].at[1]],
                                  semss[k4]).wait()

        for p in range(P):
            pltpu.sync_copy(g_hbms[p].at[pl.ds(sid * NST, NST)],
                            gsp.at[pl.ds(sid * NST, NST)])
            pltpu.sync_copy(zrows_hbm, bufs[0])
            for k in range(ZSL):
                pltpu.sync_copy(bufs[0], acc.at[pl.ds((sid * ZSL + k) * BB, BB)])
            plsc.subcore_barrier()

            pltpu.sync_copy(idx_hbm.at[base], ibs[0])
            pltpu.async_copy(idx_hbm.at[base + 1], ibs[1], semis[1])
            pltpu.async_copy(idx_hbm.at[base + 2], ibs[2], semis[2])
            gat(0, 0, 0).start()

            def step(j, k8):
                k4 = k8 % 4
                nk8, nk4 = (k8 + 1) % 8, (k8 + 1) % 4
                nxt = j + 1

                @pl.when(jnp.logical_and(nxt < nb, j >= 3))
                def _():
                    sca_wait(nk4, (k8 + 5) % 8)

                @pl.when(nxt < nb)
                def _():
                    pltpu.make_async_copy(idx_hbm.at[base + nxt], ibs[nk8],
                                          semis[nk8]).wait()
                    gat(nxt, nk4, nk8).start()

                gat(j, k4, k8).wait()

                @pl.when(j + 3 < nb)
                def _():
                    pltpu.async_copy(idx_hbm.at[base + j + 3],
                                     ibs[(k8 + 3) % 8], semis[(k8 + 3) % 8])

                sca_start(k4, k8)

            def body(j, carry):
                for k in range(8):
                    @pl.when(j % 8 == k)
                    def _(k=k):
                        step(j, k)

                return carry

            lax.fori_loop(0, nb, body, 0)
            for t in range(nb - 4, nb):
                sca_wait(t % 4, t % 8)
            plsc.subcore_barrier()
            pltpu.sync_copy(acc.at[pl.ds(sid * R_DUMP, R_DUMP)],
                            part_hbms[p].at[cid, pl.ds(sid * R_DUMP, R_DUMP)])
            if p + 1 < P:
                plsc.subcore_barrier()

    return _scatter


TB = NB * NW
_sc_scatter64 = _make_scatter_spmem(1, NB, NB)
_sc_scatter128 = _make_scatter_spmem(2, NB, NB)


def _dinv_from(degT_ref):
    deg = degT_ref[...].sum(axis=1, keepdims=True) + 1.0
    return lax.rsqrt(jnp.maximum(deg, 1.0))


def _tc1a_body(x_ref, W1_ref, mm_ref):
    mm_ref[...] = jnp.dot(x_ref[...], W1_ref[...],
                          preferred_element_type=jnp.float32)


def _tc1b_body(degT_ref, mm_ref, g1_ref):
    g1_ref[...] = mm_ref[...] * _dinv_from(degT_ref)


def _tc2_body(degT_ref, p_ref, g1_ref, b1_ref, W2_ref, g2a_ref, g2b_ref):
    dinv = _dinv_from(degT_ref)
    s = p_ref[0] + p_ref[1] + g1_ref[...]
    h = jnp.maximum(dinv * s + b1_ref[...], 0.0)
    g2 = jnp.dot(h, W2_ref[...], preferred_element_type=jnp.float32) * dinv
    g2a_ref[...] = g2[:, :D_HID]
    g2b_ref[...] = g2[:, D_HID:]


def _tc3_body(degT_ref, qa_ref, qb_ref, g2a_ref, g2b_ref, b2_ref, out_ref):
    dinv = _dinv_from(degT_ref)
    ya = dinv * (qa_ref[0] + qa_ref[1] + g2a_ref[...])
    yb = dinv * (qb_ref[0] + qb_ref[1] + g2b_ref[...])
    out_ref[...] = jnp.concatenate([ya, yb], axis=1) + b2_ref[...]


_GRID = (N // ROW_BLK,)

_tc1a = pl.pallas_call(
    _tc1a_body,
    grid=_GRID,
    in_specs=[
        pl.BlockSpec((ROW_BLK, D_IN), lambda i: (i, 0)),
        pl.BlockSpec((D_IN, D_HID), lambda i: (0, 0)),
    ],
    out_specs=pl.BlockSpec((ROW_BLK, D_HID), lambda i: (i, 0)),
    out_shape=jax.ShapeDtypeStruct((N, D_HID), jnp.float32),
)

_tc1b = pl.pallas_call(
    _tc1b_body,
    grid=_GRID,
    in_specs=[
        pl.BlockSpec((ROW_BLK, 2), lambda i: (i, 0)),
        pl.BlockSpec((ROW_BLK, D_HID), lambda i: (i, 0)),
    ],
    out_specs=pl.BlockSpec((ROW_BLK, D_HID), lambda i: (i, 0)),
    out_shape=jax.ShapeDtypeStruct((N, D_HID), jnp.float32),
)

_tc2 = pl.pallas_call(
    _tc2_body,
    grid=_GRID,
    in_specs=[
        pl.BlockSpec((ROW_BLK, 2), lambda i: (i, 0)),
        pl.BlockSpec((NC, ROW_BLK, D_HID), lambda i: (0, i, 0)),
        pl.BlockSpec((ROW_BLK, D_HID), lambda i: (i, 0)),
        pl.BlockSpec((1, D_HID), lambda i: (0, 0)),
        pl.BlockSpec((D_HID, D_IN), lambda i: (0, 0)),
    ],
    out_specs=[
        pl.BlockSpec((ROW_BLK, D_HID), lambda i: (i, 0)),
        pl.BlockSpec((ROW_BLK, D_HID), lambda i: (i, 0)),
    ],
    out_shape=[
        jax.ShapeDtypeStruct((N, D_HID), jnp.float32),
        jax.ShapeDtypeStruct((N, D_HID), jnp.float32),
    ],
)

_tc3 = pl.pallas_call(
    _tc3_body,
    grid=_GRID,
    in_specs=[
        pl.BlockSpec((ROW_BLK, 2), lambda i: (i, 0)),
        pl.BlockSpec((NC, ROW_BLK, D_HID), lambda i: (0, i, 0)),
        pl.BlockSpec((NC, ROW_BLK, D_HID), lambda i: (0, i, 0)),
        pl.BlockSpec((ROW_BLK, D_HID), lambda i: (i, 0)),
        pl.BlockSpec((ROW_BLK, D_HID), lambda i: (i, 0)),
        pl.BlockSpec((1, D_IN), lambda i: (0, 0)),
    ],
    out_specs=pl.BlockSpec((ROW_BLK, D_IN), lambda i: (i, 0)),
    out_shape=jax.ShapeDtypeStruct((N, D_IN), jnp.float32),
)


def kernel(x, edge_index, W1, b1, W2, b2):
    ei = edge_index.astype(jnp.int32)
    ei_p = jnp.pad(ei, ((0, 0), (0, E_PAD - E)), constant_values=N)
    idx = ei_p.reshape(2, TB, BB).transpose(1, 0, 2)
    ones = jnp.ones((BB,), jnp.float32)
    zeros = jnp.zeros((BB,), jnp.float32)
    z64 = jnp.zeros((BB, D_HID), jnp.float32)

    deg2 = _sc_degree(idx, ones, zeros)
    mm1 = _tc1a(x, W1)
    degT = deg2.T
    g1 = _tc1b(degT, mm1)
    (p,) = _sc_scatter64(g1, idx, z64)
    g2a, g2b = _tc2(degT, p, g1, b1.reshape(1, -1), W2)
    qa, qb = _sc_scatter128(g2a, g2b, idx, z64)
    out = _tc3(degT, qa, qb, g2a, g2b, b2.reshape(1, -1))
    return out


# speedup vs baseline: 1.6980x; 1.0097x over previous
import functools

import jax
import jax.numpy as jnp
from jax import lax
from jax.experimental import pallas as pl
from jax.experimental.pallas import tpu as pltpu
from jax.experimental.pallas import tpu_sc as plsc

N = 10000
E = 320000
D_IN = 128
D_HID = 64

NC = 2
NS = 16
NW = NC * NS
BB = 128
NB = 79
EPT = NB * BB
E_PAD = EPT * NW
N_ACC = 10240
ZSL = N_ACC // BB // NS
R_DUMP = N_ACC // NS
ROW_BLK = 2000


def _mesh():
    return plsc.VectorSubcoreMesh(core_axis_name="c", subcore_axis_name="s")


@functools.partial(
    pl.kernel,
    out_type=jax.ShapeDtypeStruct((NC, N_ACC), jnp.float32),
    mesh=_mesh(),
    compiler_params=pltpu.CompilerParams(use_tc_tiling_on_sc=False),
    scratch_types=[
        pltpu.VMEM((NB, 2, BB), jnp.int32),
        pltpu.VMEM((BB,), jnp.float32),
        pltpu.VMEM((BB,), jnp.float32),
        pltpu.VMEM_SHARED((N_ACC,), jnp.float32),
        pltpu.SemaphoreType.DMA,
        pltpu.SemaphoreType.DMA,
        pltpu.SemaphoreType.DMA,
        pltpu.SemaphoreType.DMA,
    ],
)
def _sc_degree(idx_hbm, ones_hbm, zeros_hbm, deg_hbm, idx_v, ones_v, zeros_v,
               acc, sd0, sd1, sd2, sd3):
    semds = (sd0, sd1, sd2, sd3)
    cid = lax.axis_index("c")
    sid = lax.axis_index("s")
    wid = sid * NC + cid
    pltpu.sync_copy(idx_hbm.at[pl.ds(wid * NB, NB)], idx_v)
    pltpu.sync_copy(ones_hbm, ones_v)
    pltpu.sync_copy(zeros_hbm, zeros_v)
    for k in range(ZSL):
        pltpu.sync_copy(zeros_v, acc.at[pl.ds((sid * ZSL + k) * BB, BB)])
    plsc.subcore_barrier()

    def body(j, carry):
        for k in range(4):
            @pl.when(j % 4 == k)
            def _(k=k):
                @pl.when(j >= 4)
                def _():
                    pltpu.make_async_copy(ones_v, acc.at[idx_v.at[j - 4, 1]],
                                          semds[k]).wait()
                pltpu.async_copy(ones_v, acc.at[idx_v.at[j, 1]], semds[k],
                                 add=True)

        return carry

    lax.fori_loop(0, NB, body, 0)
    for t in range(NB - 4, NB):
        pltpu.make_async_copy(ones_v, acc.at[idx_v.at[t, 1]],
                              semds[t % 4]).wait()
    plsc.subcore_barrier()
    pltpu.sync_copy(acc.at[pl.ds(sid * R_DUMP, R_DUMP)],
                    deg_hbm.at[cid, pl.ds(sid * R_DUMP, R_DUMP)])


def _make_scatter_spmem(P, nb0, nb1):
    DW = D_HID
    NST = N // NS
    out_types = [jax.ShapeDtypeStruct((NC, N_ACC, DW), jnp.float32)
                 for _ in range(P)]

    @functools.partial(
        pl.kernel,
        out_type=tuple(out_types),
        mesh=_mesh(),
        compiler_params=pltpu.CompilerParams(use_tc_tiling_on_sc=False),
        scratch_types=(
            [pltpu.VMEM((2, BB), jnp.int32)] * 8
            + [pltpu.VMEM((BB, DW), jnp.float32)] * 4
            + [
                pltpu.VMEM_SHARED((N + 16, DW), jnp.float32),
                pltpu.VMEM_SHARED((N_ACC, DW), jnp.float32),
            ]
            + [pltpu.SemaphoreType.DMA] * 16
        ),
    )
    def _scatter(*refs):
        g_hbms = refs[:P]
        idx_hbm, zrows_hbm = refs[P], refs[P + 1]
        part_hbms = refs[P + 2:2 * P + 2]
        scr = refs[2 * P + 2:]
        ibs = scr[0:8]
        bufs = scr[8:12]
        gsp, acc = scr[12], scr[13]
        semis = scr[14:22]
        semgs = scr[22:26]
        semss = scr[26:30]
        cid = lax.axis_index("c")
        sid = lax.axis_index("s")
        nb = NB
        base = (cid * NS + sid) * NB

        def gat(b, k4, k8):
            return pltpu.make_async_copy(gsp.at[ibs[k8].at[0]], bufs[k4],
                                         semgs[k4])

        def sca_start(k4, k8):
            pltpu.async_copy(bufs[k4], acc.at[ibs[k8].at[1]], semss[k4],
                             add=True)

        def sca_wait(k4, k8):
            pltpu.make_async_copy(bufs[k4], acc.at[ibs[k8].at[1]],
                                  semss[k4]).wait()

        for p in range(P):
            pltpu.sync_copy(g_hbms[p].at[pl.ds(sid * NST, NST)],
                            gsp.at[pl.ds(sid * NST, NST)])
            pltpu.sync_copy(zrows_hbm, bufs[0])
            for k in range(ZSL):
                pltpu.sync_copy(bufs[0], acc.at[pl.ds((sid * ZSL + k) * BB, BB)])
            plsc.subcore_barrier()

            pltpu.sync_copy(idx_hbm.at[base], ibs[0])
            pltpu.async_copy(idx_hbm.at[base + 1], ibs[1], semis[1])
            pltpu.async_copy(idx_hbm.at[base + 2], ibs[2], semis[2])
            gat(0, 0, 0).start()

            def step(j, k8):
                k4 = k8 % 4
                nk8, nk4 = (k8 + 1) % 8, (k8 + 1) % 4
                nxt = j + 1

                @pl.when(jnp.logical_and(nxt < nb, j >= 3))
                def _():
                    sca_wait(nk4, (k8 + 5) % 8)

                @pl.when(nxt < nb)
                def _():
                    pltpu.make_async_copy(idx_hbm.at[base + nxt], ibs[nk8],
                                          semis[nk8]).wait()
                    gat(nxt, nk4, nk8).start()

                gat(j, k4, k8).wait()

                @pl.when(j + 3 < nb)
                def _():
                    pltpu.async_copy(idx_hbm.at[base + j + 3],
                                     ibs[(k8 + 3) % 8], semis[(k8 + 3) % 8])

                sca_start(k4, k8)

            def body(j, carry):
                for k in range(8):
                    @pl.when(j % 8 == k)
                    def _(k=k):
                        step(j, k)

                return carry

            lax.fori_loop(0, nb, body, 0)
            for t in range(nb - 4, nb):
                sca_wait(t % 4, t % 8)
            plsc.subcore_barrier()
            pltpu.sync_copy(acc.at[pl.ds(sid * R_DUMP, R_DUMP)],
                            part_hbms[p].at[cid, pl.ds(sid * R_DUMP, R_DUMP)])
            if p + 1 < P:
                plsc.subcore_barrier()

    return _scatter


TB = NB * NW
_sc_scatter64 = _make_scatter_spmem(1, NB, NB)
_sc_scatter128 = _make_scatter_spmem(2, NB, NB)


def _dinv_from(degT_ref):
    deg = degT_ref[...].sum(axis=1, keepdims=True) + 1.0
    return lax.rsqrt(jnp.maximum(deg, 1.0))


def _tc1a_body(x_ref, W1_ref, mm_ref):
    mm_ref[...] = jnp.dot(x_ref[...], W1_ref[...],
                          preferred_element_type=jnp.float32)


def _tc1b_body(degT_ref, mm_ref, g1_ref):
    g1_ref[...] = mm_ref[...] * _dinv_from(degT_ref)


def _tc2_body(degT_ref, p_ref, g1_ref, b1_ref, W2_ref, g2a_ref, g2b_ref):
    dinv = _dinv_from(degT_ref)
    s = p_ref[0] + p_ref[1] + g1_ref[...]
    h = jnp.maximum(dinv * s + b1_ref[...], 0.0)
    g2 = jnp.dot(h, W2_ref[...], preferred_element_type=jnp.float32) * dinv
    g2a_ref[...] = g2[:, :D_HID]
    g2b_ref[...] = g2[:, D_HID:]


def _tc3_body(degT_ref, qa_ref, qb_ref, g2a_ref, g2b_ref, b2_ref, out_ref):
    dinv = _dinv_from(degT_ref)
    ya = dinv * (qa_ref[0] + qa_ref[1] + g2a_ref[...])
    yb = dinv * (qb_ref[0] + qb_ref[1] + g2b_ref[...])
    out_ref[...] = jnp.concatenate([ya, yb], axis=1) + b2_ref[...]


_GRID = (N // ROW_BLK,)

_tc1a = pl.pallas_call(
    _tc1a_body,
    grid=_GRID,
    in_specs=[
        pl.BlockSpec((ROW_BLK, D_IN), lambda i: (i, 0)),
        pl.BlockSpec((D_IN, D_HID), lambda i: (0, 0)),
    ],
    out_specs=pl.BlockSpec((ROW_BLK, D_HID), lambda i: (i, 0)),
    out_shape=jax.ShapeDtypeStruct((N, D_HID), jnp.float32),
)

_tc1b = pl.pallas_call(
    _tc1b_body,
    grid=_GRID,
    in_specs=[
        pl.BlockSpec((ROW_BLK, 2), lambda i: (i, 0)),
        pl.BlockSpec((ROW_BLK, D_HID), lambda i: (i, 0)),
    ],
    out_specs=pl.BlockSpec((ROW_BLK, D_HID), lambda i: (i, 0)),
    out_shape=jax.ShapeDtypeStruct((N, D_HID), jnp.float32),
)

_tc2 = pl.pallas_call(
    _tc2_body,
    grid=_GRID,
    in_specs=[
        pl.BlockSpec((ROW_BLK, 2), lambda i: (i, 0)),
        pl.BlockSpec((NC, ROW_BLK, D_HID), lambda i: (0, i, 0)),
        pl.BlockSpec((ROW_BLK, D_HID), lambda i: (i, 0)),
        pl.BlockSpec((1, D_HID), lambda i: (0, 0)),
        pl.BlockSpec((D_HID, D_IN), lambda i: (0, 0)),
    ],
    out_specs=[
        pl.BlockSpec((ROW_BLK, D_HID), lambda i: (i, 0)),
        pl.BlockSpec((ROW_BLK, D_HID), lambda i: (i, 0)),
    ],
    out_shape=[
        jax.ShapeDtypeStruct((N, D_HID), jnp.float32),
        jax.ShapeDtypeStruct((N, D_HID), jnp.float32),
    ],
)

_tc3 = pl.pallas_call(
    _tc3_body,
    grid=_GRID,
    in_specs=[
        pl.BlockSpec((ROW_BLK, 2), lambda i: (i, 0)),
        pl.BlockSpec((NC, ROW_BLK, D_HID), lambda i: (0, i, 0)),
        pl.BlockSpec((NC, ROW_BLK, D_HID), lambda i: (0, i, 0)),
        pl.BlockSpec((ROW_BLK, D_HID), lambda i: (i, 0)),
        pl.BlockSpec((ROW_BLK, D_HID), lambda i: (i, 0)),
        pl.BlockSpec((1, D_IN), lambda i: (0, 0)),
    ],
    out_specs=pl.BlockSpec((ROW_BLK, D_IN), lambda i: (i, 0)),
    out_shape=jax.ShapeDtypeStruct((N, D_IN), jnp.float32),
)


def kernel(x, edge_index, W1, b1, W2, b2):
    ei = edge_index.astype(jnp.int32)
    ei_p = jnp.pad(ei, ((0, 0), (0, E_PAD - E)), constant_values=N)
    idx = ei_p.reshape(2, TB, BB).transpose(1, 0, 2)
    ones = jnp.ones((BB,), jnp.float32)
    zeros = jnp.zeros((BB,), jnp.float32)
    z64 = jnp.zeros((BB, D_HID), jnp.float32)

    deg2 = _sc_degree(idx, ones, zeros)
    mm1 = _tc1a(x, W1)
    degT = deg2.T
    g1 = _tc1b(degT, mm1)
    (p,) = _sc_scatter64(g1, idx, z64)
    g2a, g2b = _tc2(degT, p, g1, b1.reshape(1, -1), W2)
    qa, qb = _sc_scatter128(g2a, g2b, idx, z64)
    out = _tc3(degT, qa, qb, g2a, g2b, b2.reshape(1, -1))
    return out
